# baseline (speedup 1.0000x reference)
import jax
import jax.numpy as jnp
from jax.experimental import pallas as pl

_NEG_SLOPE = 0.2


def _gat_dense_kernel(x_ref, adj_ref, W_ref, na2_ref, out_ref):
    N = adj_ref.shape[0]
    d = W_ref.shape[1]
    h = jnp.dot(x_ref[...], W_ref[...], preferred_element_type=jnp.float32)
    ns = jnp.dot(h, na2_ref[:, 0])
    nt = jnp.dot(h, na2_ref[:, 1])
    u = ns[:, None] + nt[None, :]
    arg = jnp.minimum(u, _NEG_SLOPE * u)
    e = jnp.where(adj_ref[...] != 0, jnp.exp(arg), 0.0)
    h1 = jnp.concatenate([h, jnp.ones((N, 1), jnp.float32)], axis=1)
    num1 = jnp.dot(e, h1, preferred_element_type=jnp.float32)
    hp = num1[:, :d] * (1.0 / num1[:, d:])
    hp = jnp.where(jnp.isnan(hp), 0.0, hp)
    out_ref[...] = jnp.where(hp > 0, hp, jnp.exp(jnp.minimum(hp, 0.0)) - 1.0)


def kernel(input, adj, W, a):
    B, N, d_in = input.shape
    d_out = W.shape[1]
    x2 = input.reshape(B * N, d_in)
    adj2 = adj.reshape(B * N, N)
    na2 = -jnp.stack([a[0, :d_out], a[0, d_out:]], axis=1)
    out = pl.pallas_call(
        _gat_dense_kernel,
        out_shape=jax.ShapeDtypeStruct((B * N, d_out), jnp.float32),
    )(x2, adj2, W, na2)
    return out.reshape(B, N, d_out)


# speedup vs baseline: 1.1540x; 1.1540x over previous
import jax
import jax.numpy as jnp
from jax.experimental import pallas as pl
from jax.experimental.pallas import tpu as pltpu

_NEG_SLOPE = 0.2


def _gat_dense_kernel(x_ref, adj_ref, W_ref, a_ref, out_ref):
    h = jnp.dot(x_ref[...], W_ref[...], preferred_element_type=jnp.float32)
    d = W_ref.shape[1]
    a_src = a_ref[0, :d]
    a_dst = a_ref[0, d:]
    ns = jnp.dot(h, -a_src)
    nt = jnp.dot(h, -a_dst)
    u = ns[:, None] + nt[None, :]
    arg = jnp.minimum(u, _NEG_SLOPE * u)
    e = jnp.where(adj_ref[...] != 0, jnp.exp(arg), 0.0)
    rowsum = jnp.sum(e, axis=1, keepdims=True)
    num = jnp.dot(e, h, preferred_element_type=jnp.float32)
    hp = num / rowsum
    hp = jnp.where(jnp.isnan(hp), 0.0, hp)
    out_ref[...] = jnp.where(hp > 0, hp, jnp.exp(jnp.minimum(hp, 0.0)) - 1.0)


def kernel(input, adj, W, a):
    B, N, d_in = input.shape
    d_out = W.shape[1]
    x2 = input.reshape(B * N, d_in)
    adj2 = adj.reshape(B * N, N)
    out = pl.pallas_call(
        _gat_dense_kernel,
        out_shape=jax.ShapeDtypeStruct((B * N, d_out), jnp.float32),
    )(x2, adj2, W, a)
    return out.reshape(B, N, d_out)
